# baseline (speedup 1.0000x reference)
import functools

import jax
import jax.numpy as jnp
from jax import lax
from jax.experimental import pallas as pl
from jax.experimental.pallas import tpu as pltpu
from jax.experimental.pallas import tpu_sc as plsc

N = 10000
E = 160000
D = 128
N_REL = 13
NTAB = N_REL

NC = 2
NS = 16
NW = NC * NS
K = 128
NCHUNK = 1280
EP = NCHUNK * K
CPT = 80
NBUF = 2
IBUF = 4
NPAD = 10112
ROWS_PER_TILE_ACC = NPAD // NS
ROWS_PER_TILE_OUT = N // NS

BN = 400
GRID_N = N // BN


def _table_body(x_ref, w_ref, b_ref, bases_ref, coef_ref, self_ref, tab_ref):
    x = x_ref[...]
    self_ref[...] = (
        jnp.dot(x, w_ref[...], preferred_element_type=jnp.float32) + b_ref[...]
    )
    h0 = jnp.dot(x, bases_ref[0], preferred_element_type=jnp.float32)
    h1 = jnp.dot(x, bases_ref[1], preferred_element_type=jnp.float32)
    for r in range(N_REL):
        tab_ref[r] = coef_ref[r, 0] * h0 + coef_ref[r, 1] * h1


def _build_table(x, W_self, b_self, bases, coefficients):
    return pl.pallas_call(
        _table_body,
        grid=(GRID_N,),
        in_specs=[
            pl.BlockSpec((BN, D), lambda i: (i, 0)),
            pl.BlockSpec((D, D), lambda i: (0, 0)),
            pl.BlockSpec((1, D), lambda i: (0, 0)),
            pl.BlockSpec((2, D, D), lambda i: (0, 0, 0)),
            pl.BlockSpec(memory_space=pltpu.SMEM),
        ],
        out_specs=[
            pl.BlockSpec((BN, D), lambda i: (i, 0)),
            pl.BlockSpec((NTAB, BN, D), lambda i: (0, i, 0)),
        ],
        out_shape=[
            jax.ShapeDtypeStruct((N, D), jnp.float32),
            jax.ShapeDtypeStruct((NTAB, N, D), jnp.float32),
        ],
    )(x, W_self, b_self.reshape(1, D), bases, coefficients)


def _gidx_body(src_ref, et_ref, out_ref):
    out_ref[...] = et_ref[...] * N + src_ref[...]


def _build_gidx(src_p, et_p):
    return pl.pallas_call(
        _gidx_body,
        grid=(10,),
        in_specs=[
            pl.BlockSpec((EP // K // 10, K), lambda i: (i, 0)),
            pl.BlockSpec((EP // K // 10, K), lambda i: (i, 0)),
        ],
        out_specs=pl.BlockSpec((EP // K // 10, K), lambda i: (i, 0)),
        out_shape=jax.ShapeDtypeStruct((EP // K, K), jnp.int32),
    )(src_p, et_p)


def _sc_body(htab_hbm, gidx_hbm, tgt_hbm, out_hbm,
             idx_v, tgt_v, rows_v, acc, sem_i, sem_t, sem_g, sem_s):
    core = lax.axis_index("c")
    sub = lax.axis_index("s")

    @pl.loop(0, K)
    def _(i):
        for j in range(D // 16):
            rows_v[0, i, pl.ds(j * 16, 16)] = jnp.zeros((16,), jnp.float32)

    for k in range(ROWS_PER_TILE_ACC // K):
        pltpu.sync_copy(rows_v.at[0],
                        acc.at[pl.ds(sub * ROWS_PER_TILE_ACC + k * K, K)])
    rem = ROWS_PER_TILE_ACC % K
    pltpu.sync_copy(rows_v.at[0, pl.ds(0, rem)],
                    acc.at[pl.ds(sub * ROWS_PER_TILE_ACC
                                 + (ROWS_PER_TILE_ACC // K) * K, rem)])

    plsc.subcore_barrier()

    m = jnp.where(core == 0, CPT, 0)
    base = sub * CPT

    @pl.when(core == 0)
    def _():
        for p in range(3):
            pltpu.async_copy(gidx_hbm.at[pl.ds(base + p, 1)],
                             idx_v.at[pl.ds(p, 1)], sem_i.at[p])
            pltpu.async_copy(tgt_hbm.at[pl.ds(base + p, 1)],
                             tgt_v.at[pl.ds(p, 1)], sem_t.at[p])
        pltpu.make_async_copy(gidx_hbm.at[pl.ds(base, 1)],
                              idx_v.at[pl.ds(0, 1)], sem_i.at[0]).wait()
        pltpu.async_copy(htab_hbm.at[idx_v.at[0]], rows_v.at[0], sem_g.at[0])

    @pl.loop(0, m, step=4)
    def _(c0):
        for u in range(4):
            c = c0 + u
            b = u % NBUF
            b1 = (b + 1) % NBUF
            ib = u % IBUF
            pltpu.make_async_copy(htab_hbm.at[idx_v.at[ib]],
                                  rows_v.at[b], sem_g.at[b]).wait()
            pltpu.make_async_copy(tgt_hbm.at[pl.ds(base + c, 1)],
                                  tgt_v.at[pl.ds(ib, 1)],
                                  sem_t.at[ib]).wait()
            pltpu.async_copy(rows_v.at[b], acc.at[tgt_v.at[ib]],
                             sem_s.at[b], add=True)

            @pl.when(c >= 1)
            def _():
                ibp = (ib + IBUF - 1) % IBUF
                pltpu.make_async_copy(rows_v.at[b1],
                                      acc.at[tgt_v.at[ibp]],
                                      sem_s.at[b1]).wait()

            @pl.when(c + 3 < m)
            def _():
                ibn = (ib + 3) % IBUF
                pltpu.async_copy(gidx_hbm.at[pl.ds(base + c + 3, 1)],
                                 idx_v.at[pl.ds(ibn, 1)], sem_i.at[ibn])
                pltpu.async_copy(tgt_hbm.at[pl.ds(base + c + 3, 1)],
                                 tgt_v.at[pl.ds(ibn, 1)], sem_t.at[ibn])

            @pl.when(c + 1 < m)
            def _():
                ibn = (ib + 1) % IBUF
                pltpu.make_async_copy(gidx_hbm.at[pl.ds(base + c + 1, 1)],
                                      idx_v.at[pl.ds(ibn, 1)],
                                      sem_i.at[ibn]).wait()
                pltpu.async_copy(htab_hbm.at[idx_v.at[ibn]],
                                 rows_v.at[b1], sem_g.at[b1])

    @pl.when(core == 0)
    def _():
        pltpu.make_async_copy(rows_v.at[1], acc.at[tgt_v.at[3]],
                              sem_s.at[1]).wait()

    plsc.subcore_barrier()

    @pl.when((core == 0) & (sub < NS - 1))
    def _():
        pltpu.sync_copy(acc.at[pl.ds(sub * 624, 624)],
                        out_hbm.at[0, pl.ds(sub * 624, 624), :])

    @pl.when((core == 0) & (sub == NS - 1))
    def _():
        pltpu.sync_copy(acc.at[pl.ds((NS - 1) * 624, N - (NS - 1) * 624)],
                        out_hbm.at[0, pl.ds((NS - 1) * 624,
                                            N - (NS - 1) * 624), :])


def _sc_scatter(htab2d, gidx_p, tgt_p):
    mesh = plsc.VectorSubcoreMesh(core_axis_name="c", subcore_axis_name="s",
                                  num_cores=NC, num_subcores=NS)
    kfn = pl.kernel(
        _sc_body,
        out_type=jax.ShapeDtypeStruct((1, N, D), jnp.float32),
        mesh=mesh,
        scratch_types=[
            pltpu.VMEM((IBUF, K), jnp.int32),
            pltpu.VMEM((IBUF, K), jnp.int32),
            pltpu.VMEM((NBUF, K, D), jnp.float32),
            pltpu.VMEM_SHARED((NPAD, D), jnp.float32),
            pltpu.SemaphoreType.DMA((IBUF,)),
            pltpu.SemaphoreType.DMA((IBUF,)),
            pltpu.SemaphoreType.DMA((NBUF,)),
            pltpu.SemaphoreType.DMA((NBUF,)),
        ],
    )
    return kfn(htab2d, gidx_p, tgt_p)


def _combine_body(h_ref, p_ref, out_ref):
    o = h_ref[...] + p_ref[0]
    out_ref[...] = jnp.maximum(o, 0.0)


def _combine(out_self, partials):
    return pl.pallas_call(
        _combine_body,
        grid=(GRID_N,),
        in_specs=[
            pl.BlockSpec((BN, D), lambda i: (i, 0)),
            pl.BlockSpec((1, BN, D), lambda i: (0, i, 0)),
        ],
        out_specs=pl.BlockSpec((BN, D), lambda i: (i, 0)),
        out_shape=jax.ShapeDtypeStruct((N, D), jnp.float32),
    )(out_self, partials)


def kernel(node_features, edge_index, edge_type, W_self, b_self, bases,
           coefficients):
    out_self, htab = _build_table(node_features, W_self, b_self, bases,
                                  coefficients)

    src = edge_index[0].astype(jnp.int32)
    tgt = edge_index[1].astype(jnp.int32)
    et = edge_type.astype(jnp.int32)
    pad = EP - E
    src_p = jnp.concatenate([src, jnp.zeros((pad,), jnp.int32)]).reshape(EP // K, K)
    et_p = jnp.concatenate([et, jnp.zeros((pad,), jnp.int32)]).reshape(EP // K, K)
    pad_tgt = N + (jnp.arange(pad, dtype=jnp.int32) % (NPAD - N))
    tgt_p = jnp.concatenate([tgt, pad_tgt]).reshape(EP // K, K)

    gidx_p = _build_gidx(src_p, et_p)
    partials = _sc_scatter(htab.reshape(N * NTAB, D), gidx_p, tgt_p)
    return _combine(out_self, partials)


# speedup vs baseline: 1.8463x; 1.8463x over previous
import functools

import jax
import jax.numpy as jnp
from jax import lax
from jax.experimental import pallas as pl
from jax.experimental.pallas import tpu as pltpu
from jax.experimental.pallas import tpu_sc as plsc

N = 10000
E = 160000
D = 128
N_REL = 13
NTAB = N_REL

NC = 2
NS = 16
NW = NC * NS
K = 128
NCHUNK = 1280
EP = NCHUNK * K
CPT = 80
NBUF = 2
IBUF = 4
NPAD = 10112
ROWS_PER_TILE_ACC = NPAD // NS
ROWS_PER_TILE_OUT = N // NS

BN = 400
GRID_N = N // BN


def _table_body(x_ref, w_ref, b_ref, bases_ref, coef_ref, self_ref, tab_ref):
    x = x_ref[...]
    self_ref[...] = (
        jnp.dot(x, w_ref[...], preferred_element_type=jnp.float32) + b_ref[...]
    )
    h0 = jnp.dot(x, bases_ref[0], preferred_element_type=jnp.float32)
    h1 = jnp.dot(x, bases_ref[1], preferred_element_type=jnp.float32)
    for r in range(N_REL):
        tab_ref[r] = coef_ref[r, 0] * h0 + coef_ref[r, 1] * h1


def _build_table(x, W_self, b_self, bases, coefficients):
    return pl.pallas_call(
        _table_body,
        grid=(GRID_N,),
        in_specs=[
            pl.BlockSpec((BN, D), lambda i: (i, 0)),
            pl.BlockSpec((D, D), lambda i: (0, 0)),
            pl.BlockSpec((1, D), lambda i: (0, 0)),
            pl.BlockSpec((2, D, D), lambda i: (0, 0, 0)),
            pl.BlockSpec(memory_space=pltpu.SMEM),
        ],
        out_specs=[
            pl.BlockSpec((BN, D), lambda i: (i, 0)),
            pl.BlockSpec((NTAB, BN, D), lambda i: (0, i, 0)),
        ],
        out_shape=[
            jax.ShapeDtypeStruct((N, D), jnp.float32),
            jax.ShapeDtypeStruct((NTAB, N, D), jnp.float32),
        ],
    )(x, W_self, b_self.reshape(1, D), bases, coefficients)


def _gidx_body(src_ref, et_ref, out_ref):
    out_ref[...] = et_ref[...] * N + src_ref[...]


def _build_gidx(src_p, et_p):
    return pl.pallas_call(
        _gidx_body,
        grid=(10,),
        in_specs=[
            pl.BlockSpec((EP // K // 10, K), lambda i: (i, 0)),
            pl.BlockSpec((EP // K // 10, K), lambda i: (i, 0)),
        ],
        out_specs=pl.BlockSpec((EP // K // 10, K), lambda i: (i, 0)),
        out_shape=jax.ShapeDtypeStruct((EP // K, K), jnp.int32),
    )(src_p, et_p)


def _sc_body(htab_hbm, gidx_hbm, tgt_hbm, out_hbm,
             idx_v, tgt_v, rows_v, acc, sem_i, sem_t, sem_g, sem_s):
    core = lax.axis_index("c")
    sub = lax.axis_index("s")

    @pl.loop(0, K)
    def _(i):
        for j in range(D // 16):
            rows_v[0, i, pl.ds(j * 16, 16)] = jnp.zeros((16,), jnp.float32)

    for k in range(ROWS_PER_TILE_ACC // K):
        pltpu.sync_copy(rows_v.at[0],
                        acc.at[pl.ds(sub * ROWS_PER_TILE_ACC + k * K, K)])
    rem = ROWS_PER_TILE_ACC % K
    pltpu.sync_copy(rows_v.at[0, pl.ds(0, rem)],
                    acc.at[pl.ds(sub * ROWS_PER_TILE_ACC
                                 + (ROWS_PER_TILE_ACC // K) * K, rem)])

    plsc.subcore_barrier()

    m = jnp.where(core == 0, CPT, 0)
    base = sub * CPT

    @pl.when(core == 0)
    def _():
        for p in range(3):
            pltpu.async_copy(gidx_hbm.at[pl.ds(base + p, 1)],
                             idx_v.at[pl.ds(p, 1)], sem_i.at[p])
            pltpu.async_copy(tgt_hbm.at[pl.ds(base + p, 1)],
                             tgt_v.at[pl.ds(p, 1)], sem_t.at[p])
        pltpu.make_async_copy(gidx_hbm.at[pl.ds(base, 1)],
                              idx_v.at[pl.ds(0, 1)], sem_i.at[0]).wait()
        pltpu.async_copy(htab_hbm.at[idx_v.at[0]], rows_v.at[0], sem_g.at[0])

    @pl.loop(0, m, step=4)
    def _(c0):
        for u in range(4):
            c = c0 + u
            b = u % NBUF
            b1 = (b + 1) % NBUF
            ib = u % IBUF
            pltpu.make_async_copy(htab_hbm.at[idx_v.at[ib]],
                                  rows_v.at[b], sem_g.at[b]).wait()
            pltpu.make_async_copy(tgt_hbm.at[pl.ds(base + c, 1)],
                                  tgt_v.at[pl.ds(ib, 1)],
                                  sem_t.at[ib]).wait()
            pltpu.async_copy(rows_v.at[b], acc.at[tgt_v.at[ib]],
                             sem_s.at[b], add=True)

            @pl.when(c >= 1)
            def _():
                ibp = (ib + IBUF - 1) % IBUF
                pltpu.make_async_copy(rows_v.at[b1],
                                      acc.at[tgt_v.at[ibp]],
                                      sem_s.at[b1]).wait()

            @pl.when(c + 3 < m)
            def _():
                ibn = (ib + 3) % IBUF
                pltpu.async_copy(gidx_hbm.at[pl.ds(base + c + 3, 1)],
                                 idx_v.at[pl.ds(ibn, 1)], sem_i.at[ibn])
                pltpu.async_copy(tgt_hbm.at[pl.ds(base + c + 3, 1)],
                                 tgt_v.at[pl.ds(ibn, 1)], sem_t.at[ibn])

            @pl.when(c + 1 < m)
            def _():
                ibn = (ib + 1) % IBUF
                pltpu.make_async_copy(gidx_hbm.at[pl.ds(base + c + 1, 1)],
                                      idx_v.at[pl.ds(ibn, 1)],
                                      sem_i.at[ibn]).wait()
                pltpu.async_copy(htab_hbm.at[idx_v.at[ibn]],
                                 rows_v.at[b1], sem_g.at[b1])

    @pl.when(core == 0)
    def _():
        pltpu.make_async_copy(rows_v.at[1], acc.at[tgt_v.at[3]],
                              sem_s.at[1]).wait()

    plsc.subcore_barrier()

    @pl.when((core == 0) & (sub < NS - 1))
    def _():
        pltpu.sync_copy(acc.at[pl.ds(sub * 624, 624)],
                        out_hbm.at[0, pl.ds(sub * 624, 624), :])

    @pl.when((core == 0) & (sub == NS - 1))
    def _():
        pltpu.sync_copy(acc.at[pl.ds((NS - 1) * 624, N - (NS - 1) * 624)],
                        out_hbm.at[0, pl.ds((NS - 1) * 624,
                                            N - (NS - 1) * 624), :])


def _sc_scatter(htab2d, gidx_p, tgt_p):
    mesh = plsc.VectorSubcoreMesh(core_axis_name="c", subcore_axis_name="s",
                                  num_cores=NC, num_subcores=NS)
    kfn = pl.kernel(
        _sc_body,
        out_type=jax.ShapeDtypeStruct((1, N, D), jnp.float32),
        mesh=mesh,
        scratch_types=[
            pltpu.VMEM((IBUF, K), jnp.int32),
            pltpu.VMEM((IBUF, K), jnp.int32),
            pltpu.VMEM((NBUF, K, D), jnp.float32),
            pltpu.VMEM_SHARED((NPAD, D), jnp.float32),
            pltpu.SemaphoreType.DMA((IBUF,)),
            pltpu.SemaphoreType.DMA((IBUF,)),
            pltpu.SemaphoreType.DMA((NBUF,)),
            pltpu.SemaphoreType.DMA((NBUF,)),
        ],
    )
    return kfn(htab2d, gidx_p, tgt_p)


def _combine_body(h_ref, p_ref, out_ref):
    o = h_ref[...] + p_ref[0]
    out_ref[...] = jnp.maximum(o, 0.0)


def _combine(out_self, partials):
    return pl.pallas_call(
        _combine_body,
        grid=(GRID_N,),
        in_specs=[
            pl.BlockSpec((BN, D), lambda i: (i, 0)),
            pl.BlockSpec((1, BN, D), lambda i: (0, i, 0)),
        ],
        out_specs=pl.BlockSpec((BN, D), lambda i: (i, 0)),
        out_shape=jax.ShapeDtypeStruct((N, D), jnp.float32),
    )(out_self, partials)


def kernel(node_features, edge_index, edge_type, W_self, b_self, bases,
           coefficients):
    out_self, htab = _build_table(node_features, W_self, b_self, bases,
                                  coefficients)

    src = edge_index[0].astype(jnp.int32)
    tgt = edge_index[1].astype(jnp.int32)
    et = edge_type.astype(jnp.int32)
    pad = EP - E
    pad_src = jnp.arange(pad, dtype=jnp.int32) % N
    src_p = jnp.concatenate([src, pad_src]).reshape(EP // K, K)
    et_p = jnp.concatenate([et, jnp.zeros((pad,), jnp.int32)]).reshape(EP // K, K)
    pad_tgt = N + (jnp.arange(pad, dtype=jnp.int32) % (NPAD - N))
    tgt_p = jnp.concatenate([tgt, pad_tgt]).reshape(EP // K, K)

    gidx_p = _build_gidx(src_p, et_p)
    partials = _sc_scatter(htab.reshape(N * NTAB, D), gidx_p, tgt_p)
    return _combine(out_self, partials)


# speedup vs baseline: 2.4572x; 1.3309x over previous
import functools

import jax
import jax.numpy as jnp
from jax import lax
from jax.experimental import pallas as pl
from jax.experimental.pallas import tpu as pltpu
from jax.experimental.pallas import tpu_sc as plsc

N = 10000
E = 160000
D = 128
N_REL = 13
NTAB = N_REL

NC = 2
NS = 16
NW = NC * NS
K = 128
NCHUNK = 1280
EP = NCHUNK * K
CPT = 40
NBUF = 2
IBUF = 4
NPAD = 10112
ROWS_PER_TILE_ACC = NPAD // NS
ROWS_PER_TILE_OUT = N // NS

BN = 400
GRID_N = N // BN


def _table_body(x_ref, w_ref, b_ref, bases_ref, coef_ref, self_ref, tab_ref):
    x = x_ref[...]
    self_ref[...] = (
        jnp.dot(x, w_ref[...], preferred_element_type=jnp.float32) + b_ref[...]
    )
    h0 = jnp.dot(x, bases_ref[0], preferred_element_type=jnp.float32)
    h1 = jnp.dot(x, bases_ref[1], preferred_element_type=jnp.float32)
    for r in range(N_REL):
        tab_ref[r] = coef_ref[r, 0] * h0 + coef_ref[r, 1] * h1


def _build_table(x, W_self, b_self, bases, coefficients):
    return pl.pallas_call(
        _table_body,
        grid=(GRID_N,),
        in_specs=[
            pl.BlockSpec((BN, D), lambda i: (i, 0)),
            pl.BlockSpec((D, D), lambda i: (0, 0)),
            pl.BlockSpec((1, D), lambda i: (0, 0)),
            pl.BlockSpec((2, D, D), lambda i: (0, 0, 0)),
            pl.BlockSpec(memory_space=pltpu.SMEM),
        ],
        out_specs=[
            pl.BlockSpec((BN, D), lambda i: (i, 0)),
            pl.BlockSpec((NTAB, BN, D), lambda i: (0, i, 0)),
        ],
        out_shape=[
            jax.ShapeDtypeStruct((N, D), jnp.float32),
            jax.ShapeDtypeStruct((NTAB, N, D), jnp.float32),
        ],
    )(x, W_self, b_self.reshape(1, D), bases, coefficients)


def _gidx_body(src_ref, et_ref, out_ref):
    out_ref[...] = et_ref[...] * N + src_ref[...]


def _build_gidx(src_p, et_p):
    return pl.pallas_call(
        _gidx_body,
        grid=(10,),
        in_specs=[
            pl.BlockSpec((EP // K // 10, K), lambda i: (i, 0)),
            pl.BlockSpec((EP // K // 10, K), lambda i: (i, 0)),
        ],
        out_specs=pl.BlockSpec((EP // K // 10, K), lambda i: (i, 0)),
        out_shape=jax.ShapeDtypeStruct((EP // K, K), jnp.int32),
    )(src_p, et_p)


def _sc_body(htab_hbm, gidx_hbm, tgt_hbm, out_hbm,
             idx_v, tgt_v, rows_v, acc, sem_i, sem_t, sem_g, sem_s):
    core = lax.axis_index("c")
    sub = lax.axis_index("s")

    @pl.loop(0, K)
    def _(i):
        for j in range(D // 16):
            rows_v[0, i, pl.ds(j * 16, 16)] = jnp.zeros((16,), jnp.float32)

    for k in range(ROWS_PER_TILE_ACC // K):
        pltpu.sync_copy(rows_v.at[0],
                        acc.at[pl.ds(sub * ROWS_PER_TILE_ACC + k * K, K)])
    rem = ROWS_PER_TILE_ACC % K
    pltpu.sync_copy(rows_v.at[0, pl.ds(0, rem)],
                    acc.at[pl.ds(sub * ROWS_PER_TILE_ACC
                                 + (ROWS_PER_TILE_ACC // K) * K, rem)])

    plsc.subcore_barrier()

    m = CPT
    base = (core * NS + sub) * CPT

    for p in range(3):
        pltpu.async_copy(gidx_hbm.at[pl.ds(base + p, 1)],
                         idx_v.at[pl.ds(p, 1)], sem_i.at[p])
        pltpu.async_copy(tgt_hbm.at[pl.ds(base + p, 1)],
                         tgt_v.at[pl.ds(p, 1)], sem_t.at[p])
    pltpu.make_async_copy(gidx_hbm.at[pl.ds(base, 1)],
                          idx_v.at[pl.ds(0, 1)], sem_i.at[0]).wait()
    pltpu.async_copy(htab_hbm.at[idx_v.at[0]], rows_v.at[0], sem_g.at[0])

    @pl.loop(0, m, step=4)
    def _(c0):
        for u in range(4):
            c = c0 + u
            b = u % NBUF
            b1 = (b + 1) % NBUF
            ib = u % IBUF
            pltpu.make_async_copy(htab_hbm.at[idx_v.at[ib]],
                                  rows_v.at[b], sem_g.at[b]).wait()
            pltpu.make_async_copy(tgt_hbm.at[pl.ds(base + c, 1)],
                                  tgt_v.at[pl.ds(ib, 1)],
                                  sem_t.at[ib]).wait()
            pltpu.async_copy(rows_v.at[b], acc.at[tgt_v.at[ib]],
                             sem_s.at[b], add=True)

            @pl.when(c >= 1)
            def _():
                ibp = (ib + IBUF - 1) % IBUF
                pltpu.make_async_copy(rows_v.at[b1],
                                      acc.at[tgt_v.at[ibp]],
                                      sem_s.at[b1]).wait()

            @pl.when(c + 3 < m)
            def _():
                ibn = (ib + 3) % IBUF
                pltpu.async_copy(gidx_hbm.at[pl.ds(base + c + 3, 1)],
                                 idx_v.at[pl.ds(ibn, 1)], sem_i.at[ibn])
                pltpu.async_copy(tgt_hbm.at[pl.ds(base + c + 3, 1)],
                                 tgt_v.at[pl.ds(ibn, 1)], sem_t.at[ibn])

            @pl.when(c + 1 < m)
            def _():
                ibn = (ib + 1) % IBUF
                pltpu.make_async_copy(gidx_hbm.at[pl.ds(base + c + 1, 1)],
                                      idx_v.at[pl.ds(ibn, 1)],
                                      sem_i.at[ibn]).wait()
                pltpu.async_copy(htab_hbm.at[idx_v.at[ibn]],
                                 rows_v.at[b1], sem_g.at[b1])

    pltpu.make_async_copy(rows_v.at[1], acc.at[tgt_v.at[3]],
                          sem_s.at[1]).wait()

    plsc.subcore_barrier()

    @pl.when(sub < NS - 1)
    def _():
        pltpu.sync_copy(acc.at[pl.ds(sub * 624, 624)],
                        out_hbm.at[core, pl.ds(sub * 624, 624), :])

    @pl.when(sub == NS - 1)
    def _():
        pltpu.sync_copy(acc.at[pl.ds((NS - 1) * 624, N - (NS - 1) * 624)],
                        out_hbm.at[core, pl.ds((NS - 1) * 624,
                                               N - (NS - 1) * 624), :])


def _sc_scatter(htab2d, gidx_p, tgt_p):
    mesh = plsc.VectorSubcoreMesh(core_axis_name="c", subcore_axis_name="s",
                                  num_cores=NC, num_subcores=NS)
    kfn = pl.kernel(
        _sc_body,
        out_type=jax.ShapeDtypeStruct((NC, N, D), jnp.float32),
        mesh=mesh,
        scratch_types=[
            pltpu.VMEM((IBUF, K), jnp.int32),
            pltpu.VMEM((IBUF, K), jnp.int32),
            pltpu.VMEM((NBUF, K, D), jnp.float32),
            pltpu.VMEM_SHARED((NPAD, D), jnp.float32),
            pltpu.SemaphoreType.DMA((IBUF,)),
            pltpu.SemaphoreType.DMA((IBUF,)),
            pltpu.SemaphoreType.DMA((NBUF,)),
            pltpu.SemaphoreType.DMA((NBUF,)),
        ],
    )
    return kfn(htab2d, gidx_p, tgt_p)


def _combine_body(h_ref, p_ref, out_ref):
    o = h_ref[...] + p_ref[0] + p_ref[1]
    out_ref[...] = jnp.maximum(o, 0.0)


def _combine(out_self, partials):
    return pl.pallas_call(
        _combine_body,
        grid=(GRID_N,),
        in_specs=[
            pl.BlockSpec((BN, D), lambda i: (i, 0)),
            pl.BlockSpec((NC, BN, D), lambda i: (0, i, 0)),
        ],
        out_specs=pl.BlockSpec((BN, D), lambda i: (i, 0)),
        out_shape=jax.ShapeDtypeStruct((N, D), jnp.float32),
    )(out_self, partials)


def kernel(node_features, edge_index, edge_type, W_self, b_self, bases,
           coefficients):
    out_self, htab = _build_table(node_features, W_self, b_self, bases,
                                  coefficients)

    src = edge_index[0].astype(jnp.int32)
    tgt = edge_index[1].astype(jnp.int32)
    et = edge_type.astype(jnp.int32)
    pad = EP - E
    pad_src = jnp.arange(pad, dtype=jnp.int32) % N
    src_p = jnp.concatenate([src, pad_src]).reshape(EP // K, K)
    et_p = jnp.concatenate([et, jnp.zeros((pad,), jnp.int32)]).reshape(EP // K, K)
    pad_tgt = N + (jnp.arange(pad, dtype=jnp.int32) % (NPAD - N))
    tgt_p = jnp.concatenate([tgt, pad_tgt]).reshape(EP // K, K)

    gidx_p = _build_gidx(src_p, et_p)
    partials = _sc_scatter(htab.reshape(N * NTAB, D), gidx_p, tgt_p)
    return _combine(out_self, partials)


# speedup vs baseline: 2.8118x; 1.1443x over previous
import functools

import jax
import jax.numpy as jnp
from jax import lax
from jax.experimental import pallas as pl
from jax.experimental.pallas import tpu as pltpu
from jax.experimental.pallas import tpu_sc as plsc

N = 10000
E = 160000
D = 128
N_REL = 13
NTAB = N_REL

NC = 2
NS = 16
NW = NC * NS
K = 128
NCHUNK = 1280
EP = NCHUNK * K
CPT = 40
NBUF = 2
IBUF = 4
NPAD = 10112
ROWS_PER_TILE_ACC = NPAD // NS
ROWS_PER_TILE_OUT = N // NS

BN = 1000
GRID_N = N // BN


def _table_body(x_ref, w_ref, b_ref, bases_ref, coef_ref, src_ref, et_ref,
                self_ref, tab_ref, gidx_ref):
    x = x_ref[...]
    self_ref[...] = (
        jnp.dot(x, w_ref[...], preferred_element_type=jnp.float32) + b_ref[...]
    )
    h0 = jnp.dot(x, bases_ref[0], preferred_element_type=jnp.float32)
    h1 = jnp.dot(x, bases_ref[1], preferred_element_type=jnp.float32)
    for r in range(N_REL):
        tab_ref[r] = coef_ref[r, 0] * h0 + coef_ref[r, 1] * h1
    gidx_ref[...] = et_ref[...] * N + src_ref[...]


def _build_table(x, W_self, b_self, bases, coefficients, src_p, et_p):
    eb = EP // K // GRID_N
    return pl.pallas_call(
        _table_body,
        grid=(GRID_N,),
        in_specs=[
            pl.BlockSpec((BN, D), lambda i: (i, 0)),
            pl.BlockSpec((D, D), lambda i: (0, 0)),
            pl.BlockSpec((1, D), lambda i: (0, 0)),
            pl.BlockSpec((2, D, D), lambda i: (0, 0, 0)),
            pl.BlockSpec(memory_space=pltpu.SMEM),
            pl.BlockSpec((eb, K), lambda i: (i, 0)),
            pl.BlockSpec((eb, K), lambda i: (i, 0)),
        ],
        out_specs=[
            pl.BlockSpec((BN, D), lambda i: (i, 0)),
            pl.BlockSpec((NTAB, BN, D), lambda i: (0, i, 0)),
            pl.BlockSpec((eb, K), lambda i: (i, 0)),
        ],
        out_shape=[
            jax.ShapeDtypeStruct((N, D), jnp.float32),
            jax.ShapeDtypeStruct((NTAB, N, D), jnp.float32),
            jax.ShapeDtypeStruct((EP // K, K), jnp.int32),
        ],
    )(x, W_self, b_self.reshape(1, D), bases, coefficients, src_p, et_p)


def _sc_body(htab_hbm, gidx_hbm, tgt_hbm, out_hbm,
             idx_v, tgt_v, rows_v, acc, sem_i, sem_t, sem_g, sem_s):
    core = lax.axis_index("c")
    sub = lax.axis_index("s")

    @pl.loop(0, K)
    def _(i):
        for j in range(D // 16):
            rows_v[0, i, pl.ds(j * 16, 16)] = jnp.zeros((16,), jnp.float32)

    for k in range(ROWS_PER_TILE_ACC // K):
        pltpu.sync_copy(rows_v.at[0],
                        acc.at[pl.ds(sub * ROWS_PER_TILE_ACC + k * K, K)])
    rem = ROWS_PER_TILE_ACC % K
    pltpu.sync_copy(rows_v.at[0, pl.ds(0, rem)],
                    acc.at[pl.ds(sub * ROWS_PER_TILE_ACC
                                 + (ROWS_PER_TILE_ACC // K) * K, rem)])

    plsc.subcore_barrier()

    m = CPT
    base = (core * NS + sub) * CPT

    for p in range(3):
        pltpu.async_copy(gidx_hbm.at[pl.ds(base + p, 1)],
                         idx_v.at[pl.ds(p, 1)], sem_i.at[p])
        pltpu.async_copy(tgt_hbm.at[pl.ds(base + p, 1)],
                         tgt_v.at[pl.ds(p, 1)], sem_t.at[p])
    pltpu.make_async_copy(gidx_hbm.at[pl.ds(base, 1)],
                          idx_v.at[pl.ds(0, 1)], sem_i.at[0]).wait()
    pltpu.async_copy(htab_hbm.at[idx_v.at[0]], rows_v.at[0], sem_g.at[0])

    @pl.loop(0, m, step=4)
    def _(c0):
        for u in range(4):
            c = c0 + u
            b = u % NBUF
            b1 = (b + 1) % NBUF
            ib = u % IBUF
            pltpu.make_async_copy(htab_hbm.at[idx_v.at[ib]],
                                  rows_v.at[b], sem_g.at[b]).wait()
            pltpu.make_async_copy(tgt_hbm.at[pl.ds(base + c, 1)],
                                  tgt_v.at[pl.ds(ib, 1)],
                                  sem_t.at[ib]).wait()
            pltpu.async_copy(rows_v.at[b], acc.at[tgt_v.at[ib]],
                             sem_s.at[b], add=True)

            @pl.when(c >= 1)
            def _():
                ibp = (ib + IBUF - 1) % IBUF
                pltpu.make_async_copy(rows_v.at[b1],
                                      acc.at[tgt_v.at[ibp]],
                                      sem_s.at[b1]).wait()

            @pl.when(c + 3 < m)
            def _():
                ibn = (ib + 3) % IBUF
                pltpu.async_copy(gidx_hbm.at[pl.ds(base + c + 3, 1)],
                                 idx_v.at[pl.ds(ibn, 1)], sem_i.at[ibn])
                pltpu.async_copy(tgt_hbm.at[pl.ds(base + c + 3, 1)],
                                 tgt_v.at[pl.ds(ibn, 1)], sem_t.at[ibn])

            @pl.when(c + 1 < m)
            def _():
                ibn = (ib + 1) % IBUF
                pltpu.make_async_copy(gidx_hbm.at[pl.ds(base + c + 1, 1)],
                                      idx_v.at[pl.ds(ibn, 1)],
                                      sem_i.at[ibn]).wait()
                pltpu.async_copy(htab_hbm.at[idx_v.at[ibn]],
                                 rows_v.at[b1], sem_g.at[b1])

    pltpu.make_async_copy(rows_v.at[1], acc.at[tgt_v.at[3]],
                          sem_s.at[1]).wait()

    plsc.subcore_barrier()

    @pl.when(sub < NS - 1)
    def _():
        pltpu.sync_copy(acc.at[pl.ds(sub * 624, 624)],
                        out_hbm.at[core, pl.ds(sub * 624, 624), :])

    @pl.when(sub == NS - 1)
    def _():
        pltpu.sync_copy(acc.at[pl.ds((NS - 1) * 624, N - (NS - 1) * 624)],
                        out_hbm.at[core, pl.ds((NS - 1) * 624,
                                               N - (NS - 1) * 624), :])


def _sc_scatter(htab2d, gidx_p, tgt_p):
    mesh = plsc.VectorSubcoreMesh(core_axis_name="c", subcore_axis_name="s",
                                  num_cores=NC, num_subcores=NS)
    kfn = pl.kernel(
        _sc_body,
        out_type=jax.ShapeDtypeStruct((NC, N, D), jnp.float32),
        mesh=mesh,
        scratch_types=[
            pltpu.VMEM((IBUF, K), jnp.int32),
            pltpu.VMEM((IBUF, K), jnp.int32),
            pltpu.VMEM((NBUF, K, D), jnp.float32),
            pltpu.VMEM_SHARED((NPAD, D), jnp.float32),
            pltpu.SemaphoreType.DMA((IBUF,)),
            pltpu.SemaphoreType.DMA((IBUF,)),
            pltpu.SemaphoreType.DMA((NBUF,)),
            pltpu.SemaphoreType.DMA((NBUF,)),
        ],
    )
    return kfn(htab2d, gidx_p, tgt_p)


def _combine_body(h_ref, p_ref, out_ref):
    o = h_ref[...] + p_ref[0] + p_ref[1]
    out_ref[...] = jnp.maximum(o, 0.0)


def _combine(out_self, partials):
    return pl.pallas_call(
        _combine_body,
        grid=(GRID_N,),
        in_specs=[
            pl.BlockSpec((BN, D), lambda i: (i, 0)),
            pl.BlockSpec((NC, BN, D), lambda i: (0, i, 0)),
        ],
        out_specs=pl.BlockSpec((BN, D), lambda i: (i, 0)),
        out_shape=jax.ShapeDtypeStruct((N, D), jnp.float32),
    )(out_self, partials)


def kernel(node_features, edge_index, edge_type, W_self, b_self, bases,
           coefficients):
    src = edge_index[0].astype(jnp.int32)
    tgt = edge_index[1].astype(jnp.int32)
    et = edge_type.astype(jnp.int32)
    pad = EP - E
    pad_src = jnp.arange(pad, dtype=jnp.int32) % N
    src_p = jnp.concatenate([src, pad_src]).reshape(EP // K, K)
    et_p = jnp.concatenate([et, jnp.zeros((pad,), jnp.int32)]).reshape(EP // K, K)
    pad_tgt = N + (jnp.arange(pad, dtype=jnp.int32) % (NPAD - N))
    tgt_p = jnp.concatenate([tgt, pad_tgt]).reshape(EP // K, K)

    out_self, htab, gidx_p = _build_table(node_features, W_self, b_self,
                                          bases, coefficients, src_p, et_p)
    partials = _sc_scatter(htab.reshape(N * NTAB, D), gidx_p, tgt_p)
    return _combine(out_self, partials)
